# baseline (speedup 1.0000x reference)
import functools

import jax
import jax.numpy as jnp
from jax import lax
from jax.experimental import pallas as pl
from jax.experimental.pallas import tpu as pltpu
from jax.experimental.pallas import tpu_sc as plsc

B = 16384
F = 16
NU = 1000000
NM = 100000
L = 16
NTILES = 32
CW = 8
CU = CW * 128

NWF_U = NU // 128
NWF_M = NM // 128
TAIL_U0 = NWF_U * 128
TAIL_M0 = NWF_M * 128
TAIL_U = NU - TAIL_U0
TAIL_M = NM - TAIL_M0
NCH_U = 31
NCH_M = 4
SAFE = B * F
OPAD = 256

_MESH = plsc.VectorSubcoreMesh(core_axis_name="c", subcore_axis_name="s")


def _wrange(wid, nwf):
    per = nwf // NTILES
    rem = nwf - per * NTILES
    wlo = wid * per + jnp.minimum(wid, rem)
    cnt = per + (wid < rem).astype(jnp.int32)
    whi = wlo + cnt + (wid == NTILES - 1).astype(jnp.int32)
    return wlo, whi


def _scan(ids_v, mid_v, mpos_v, wlo, whi):
    def body(g, n):
        idv = ids_v[pl.ds(g * L, L)]
        w = idv >> 7
        msk = (w >= wlo) & (w < whi)
        posv = g * L + lax.iota(jnp.int32, L)
        plsc.store_compressed(mid_v.at[pl.ds(n, L)], idv, mask=msk)
        plsc.store_compressed(mpos_v.at[pl.ds(n, L)], posv, mask=msk)
        return n + jnp.sum(msk.astype(jnp.int32))

    n = lax.fori_loop(0, B // L, body, 0)
    mid_v[pl.ds(n, L)] = jnp.full((L,), -1, jnp.int32)
    return (n + L - 1) >> 4


def _compact(mid_v, mpos_v, cmc_v, cmt_v, ngroups, lo_w, hi_w, col_base):
    def body(g, nc):
        idv = mid_v[pl.ds(g * L, L)]
        posv = mpos_v[pl.ds(g * L, L)]
        w = idv >> 7
        msk = (w >= lo_w) & (w < hi_w)
        plsc.store_compressed(cmc_v.at[pl.ds(nc, L)], idv - col_base, mask=msk)
        plsc.store_compressed(cmt_v.at[pl.ds(nc, L)], posv * F, mask=msk)
        return nc + jnp.sum(msk.astype(jnp.int32))

    nc = lax.fori_loop(0, ngroups, body, 0)
    cmc_v[pl.ds(nc, L)] = jnp.full((L,), 0, jnp.int32)
    cmt_v[pl.ds(nc, L)] = SAFE + lax.iota(jnp.int32, L)
    return (nc + L - 1) >> 4


def _extract(cmc_v, cmt_v, ngc, src_v, out_h, stage_d, stage_i, ssem,
             prev_out, row_is_id):
    STRIP_SCATTER = False

    def wait_pair(slot):
        if STRIP_SCATTER:
            return
        for k in range(2):
            pltpu.make_async_copy(stage_d.at[slot, k],
                                  out_h.at[stage_i.at[slot, k]], ssem).wait()

    def drain(j, c):
        wait_pair(j & 1)
        return c

    lax.fori_loop(0, prev_out, drain, 0)

    def ext(g):
        slot = g & 1
        colv = cmc_v[pl.ds(g * L, L)]
        tgtv = cmt_v[pl.ds(g * L, L)]
        iota = lax.iota(jnp.int32, L)
        rowv = iota >> 3
        cbase = (iota & 7) * F
        for f in range(F):
            fv = jnp.full((L,), f, jnp.int32)
            if row_is_id:
                vals = plsc.load_gather(src_v, [colv, fv])
            else:
                vals = plsc.load_gather(src_v, [fv, colv])
            plsc.store_scatter(stage_d.at[slot], [rowv, cbase + f], vals)
            plsc.store_scatter(stage_i.at[slot], [rowv, cbase + f], tgtv + f)
        if not STRIP_SCATTER:
            for k in range(2):
                pltpu.async_copy(stage_d.at[slot, k],
                                 out_h.at[stage_i.at[slot, k]], ssem)

    lim = jnp.minimum(ngc, 2)

    def abody(g, c):
        ext(g)
        return c

    def bbody(g, c):
        wait_pair(g & 1)
        ext(g)
        return c

    lax.fori_loop(0, lim, abody, 0)
    lax.fori_loop(lim, ngc, bbody, 0)
    return lim


def _phase(tab_h, ids_h, out_h, nwf, nch_max, tail0, tail_v,
           ids_v, mid_v, mpos_v, cmc_v, cmt_v, wbuf_v, stage_d, stage_i,
           dsem, ssem, wid, prev_out):
    pltpu.sync_copy(ids_h, ids_v)
    wlo, whi = _wrange(wid, nwf)

    def fire(ci, slot):
        eff = pl.multiple_of(
            jnp.minimum(wlo + CW * ci, nwf - CW) * 128, 128)
        pltpu.async_copy(tab_h.at[:, pl.ds(eff, CU)], wbuf_v.at[slot], dsem)

    fire(0, 0)
    ngroups = _scan(ids_v, mid_v, mpos_v, wlo, whi)

    def chunk_body(c, po):
        cur = c & 1
        pltpu.make_async_copy(tab_h.at[:, pl.ds(0, CU)], wbuf_v.at[cur],
                              dsem).wait()
        fire(jnp.minimum(c + 1, nch_max - 1), 1 - cur)
        c0 = wlo + CW * c
        c1 = jnp.minimum(c0 + CW, nwf)
        eff = jnp.minimum(c0, nwf - CW) * 128
        ngc = _compact(mid_v, mpos_v, cmc_v, cmt_v, ngroups, c0, c1, eff)
        return _extract(cmc_v, cmt_v, ngc, wbuf_v.at[cur], out_h,
                        stage_d, stage_i, ssem, po, False)

    prev_out = lax.fori_loop(0, nch_max, chunk_body, prev_out)
    pltpu.make_async_copy(tab_h.at[:, pl.ds(0, CU)],
                          wbuf_v.at[nch_max & 1], dsem).wait()

    ngc = _compact(mid_v, mpos_v, cmc_v, cmt_v, ngroups, nwf, nwf + 1, tail0)
    prev_out = _extract(cmc_v, cmt_v, ngc, tail_v, out_h, stage_d, stage_i,
                        ssem, prev_out, True)
    return prev_out


@functools.partial(
    pl.kernel,
    out_type=[
        jax.ShapeDtypeStruct((B * F + OPAD,), jnp.float32),
        jax.ShapeDtypeStruct((B * F + OPAD,), jnp.float32),
    ],
    mesh=_MESH,
    compiler_params=pltpu.CompilerParams(needs_layout_passes=False),
    scratch_types=[
        pltpu.VMEM((B,), jnp.int32),
        pltpu.VMEM((B + L,), jnp.int32),
        pltpu.VMEM((B + L,), jnp.int32),
        pltpu.VMEM((B + L,), jnp.int32),
        pltpu.VMEM((B + L,), jnp.int32),
        pltpu.VMEM((2, F, CU), jnp.float32),
        pltpu.VMEM((TAIL_U, 128), jnp.float32),
        pltpu.VMEM((2, 2, 128), jnp.float32),
        pltpu.VMEM((2, 2, 128), jnp.int32),
        pltpu.SemaphoreType.DMA,
        pltpu.SemaphoreType.DMA,
    ],
)
def _sc_gather(user_h, movie_h, ut_h, mt_h, tailu_h, tailm_h, uo_h, mo_h,
               ids_v, mid_v, mpos_v, cmc_v, cmt_v, wbuf_v, tail_v,
               stage_d, stage_i, dsem, ssem):
    wid = lax.axis_index("s") * 2 + lax.axis_index("c")
    pltpu.sync_copy(tailu_h, tail_v)
    fcnt = _phase(ut_h, user_h, uo_h, NWF_U, NCH_U, TAIL_U0, tail_v,
                  ids_v, mid_v, mpos_v, cmc_v, cmt_v, wbuf_v,
                  stage_d, stage_i, dsem, ssem, wid, 0)
    pltpu.sync_copy(tailm_h, tail_v.at[pl.ds(0, TAIL_M)])
    fcnt = _phase(mt_h, movie_h, mo_h, NWF_M, NCH_M, TAIL_M0, tail_v,
                  ids_v, mid_v, mpos_v, cmc_v, cmt_v, wbuf_v,
                  stage_d, stage_i, dsem, ssem, wid, fcnt)

    def drain(j, c):
        for k in range(2):
            pltpu.make_async_copy(stage_d.at[j & 1, k],
                                  uo_h.at[stage_i.at[j & 1, k]], ssem).wait()
        return c

    lax.fori_loop(0, fcnt, drain, 0)


def _mlp_body(u_ref, m_ref, w1u_ref, w1m_ref, b1_ref, w2_ref, b2_ref, o_ref):
    h = jnp.dot(u_ref[...], w1u_ref[...], preferred_element_type=jnp.float32)
    h = h + jnp.dot(m_ref[...], w1m_ref[...], preferred_element_type=jnp.float32)
    h = jnp.maximum(h + b1_ref[...], 0.0)
    o = jnp.dot(h, w2_ref[...], preferred_element_type=jnp.float32) + b2_ref[...]
    o_ref[...] = 5.5 / (1.0 + jnp.exp(-o))


def _mlp(u_pack, m_pack, w1u, w1m, b1, w2, b2):
    eye = jnp.eye(8, dtype=jnp.float32)
    return pl.pallas_call(
        _mlp_body,
        out_shape=jax.ShapeDtypeStruct((B // 8, 8), jnp.float32),
    )(u_pack, m_pack, jnp.kron(eye, w1u), jnp.kron(eye, w1m),
      jnp.tile(b1, 8)[None], jnp.kron(eye, w2), jnp.tile(b2, 8)[None])


def kernel(user, movie, u_table, m_table, W1, b1, W2, b2):
    user = user.astype(jnp.int32)
    movie = movie.astype(jnp.int32)
    pad = ((0, 0), (0, 128 - F))
    tailu = jnp.pad(u_table[TAIL_U0:], pad)
    tailm = jnp.pad(m_table[TAIL_M0:], pad)
    uo, mo = _sc_gather(user, movie, u_table.T, m_table.T, tailu, tailm)
    u_pack = uo[:B * F].reshape(B * F // 128, 128)
    m_pack = mo[:B * F].reshape(B * F // 128, 128)
    out = _mlp(u_pack, m_pack, W1[:F], W1[F:], b1, W2, b2)
    return out.reshape(B, 1)


# speedup vs baseline: 82.5887x; 82.5887x over previous
import functools

import jax
import jax.numpy as jnp
from jax import lax
from jax.experimental import pallas as pl
from jax.experimental.pallas import tpu as pltpu
from jax.experimental.pallas import tpu_sc as plsc

B = 16384
F = 16
NU = 1000000
NM = 100000
L = 16
NTILES = 32
CW = 8
CU = CW * 128

NWF_U = NU // 128
NWF_M = NM // 128
TAIL_U0 = NWF_U * 128
TAIL_M0 = NWF_M * 128
TAIL_U = NU - TAIL_U0
TAIL_M = NM - TAIL_M0
NCH_U = 31
NCH_M = 4
SAFE = B * F

_MESH = plsc.VectorSubcoreMesh(core_axis_name="c", subcore_axis_name="s")


def _wrange(wid, nwf):
    per = nwf // NTILES
    rem = nwf - per * NTILES
    wlo = wid * per + jnp.minimum(wid, rem)
    cnt = per + (wid < rem).astype(jnp.int32)
    whi = wlo + cnt + (wid == NTILES - 1).astype(jnp.int32)
    return wlo, whi


def _scan(ids_v, mid_v, mpos_v, wlo, whi):
    def body(g, n):
        idv = ids_v[pl.ds(g * L, L)]
        w = idv >> 7
        msk = (w >= wlo) & (w < whi)
        posv = g * L + lax.iota(jnp.int32, L)
        plsc.store_compressed(mid_v.at[pl.ds(n, L)], idv, mask=msk)
        plsc.store_compressed(mpos_v.at[pl.ds(n, L)], posv, mask=msk)
        return n + jnp.sum(msk.astype(jnp.int32))

    n = lax.fori_loop(0, B // L, body, 0)
    mid_v[pl.ds(n, L)] = jnp.full((L,), -1, jnp.int32)
    return (n + L - 1) >> 4


def _compact(mid_v, mpos_v, cmc_v, cmt_v, ngroups, lo_w, hi_w, col_base):
    def body(g, nc):
        idv = mid_v[pl.ds(g * L, L)]
        posv = mpos_v[pl.ds(g * L, L)]
        w = idv >> 7
        msk = (w >= lo_w) & (w < hi_w)
        plsc.store_compressed(cmc_v.at[pl.ds(nc, L)], idv - col_base, mask=msk)
        plsc.store_compressed(cmt_v.at[pl.ds(nc, L)], posv * F, mask=msk)
        return nc + jnp.sum(msk.astype(jnp.int32))

    nc = lax.fori_loop(0, ngroups, body, 0)
    cmc_v[pl.ds(nc, L)] = jnp.full((L,), 0, jnp.int32)
    cmt_v[pl.ds(nc, L)] = SAFE + lax.iota(jnp.int32, L) * F
    return (nc + L - 1) >> 4


def _extract(cmc_v, cmt_v, ngc, src_v, rows_h, pos_h, rbase, pbase,
             stage_d, stage_p, ssem, state, row_is_id):
    prev_out, k0 = state

    def wait_trio(slot):
        pltpu.make_async_copy(stage_d.at[pl.ds(slot * 256, 256)],
                              rows_h.at[pl.ds(0, 256)], ssem).wait()
        pltpu.make_async_copy(stage_p.at[pl.ds(slot * L, L)],
                              pos_h.at[pl.ds(0, L)], ssem).wait()

    def drain(j, c):
        wait_trio(j & 1)
        return c

    lax.fori_loop(0, prev_out, drain, 0)

    def ext(g):
        slot = (g - k0) & 1
        colv = cmc_v[pl.ds((g - k0) * L, L)]
        tgtv = cmt_v[pl.ds((g - k0) * L, L)]
        iota = lax.iota(jnp.int32, L)
        for f in range(F):
            fv = jnp.full((L,), f, jnp.int32)
            if row_is_id:
                vals = plsc.load_gather(src_v, [colv, fv])
            else:
                vals = plsc.load_gather(src_v, [fv, colv])
            plsc.store_scatter(stage_d, [slot * 256 + iota * F + f], vals)
        stage_p[pl.ds(slot * L, L)] = tgtv
        pltpu.async_copy(stage_d.at[pl.ds(slot * 256, 256)],
                         rows_h.at[pl.ds(rbase + g * 256, 256)], ssem)
        pltpu.async_copy(stage_p.at[pl.ds(slot * L, L)],
                         pos_h.at[pl.ds(pbase + g * L, L)], ssem)

    lim = jnp.minimum(ngc, 2)

    def abody(g, c):
        ext(g)
        return c

    def bbody(g, c):
        wait_trio((g - k0) & 1)
        ext(g)
        return c

    lax.fori_loop(k0, k0 + lim, abody, 0)
    lax.fori_loop(k0 + lim, k0 + ngc, bbody, 0)
    return (lim, k0 + ngc)


def _phase(tab_h, ids_h, rows_h, pos_h, nwf, nch_max, tail0, tail_v,
           ids_v, mid_v, mpos_v, cmc_v, cmt_v, wbuf_v, stage_d, stage_p,
           dsem, ssem, wid, state):
    pltpu.sync_copy(ids_h, ids_v)
    wlo, whi = _wrange(wid, nwf)
    rbase = wid * (B * F)
    pbase = wid * B

    def fire(ci, slot):
        eff = pl.multiple_of(
            jnp.minimum(wlo + CW * ci, nwf - CW) * 128, 128)
        pltpu.async_copy(tab_h.at[:, pl.ds(eff, CU)], wbuf_v.at[slot], dsem)

    fire(0, 0)
    ngroups = _scan(ids_v, mid_v, mpos_v, wlo, whi)

    def chunk_body(c, st):
        cur = c & 1
        pltpu.make_async_copy(tab_h.at[:, pl.ds(0, CU)], wbuf_v.at[cur],
                              dsem).wait()
        fire(jnp.minimum(c + 1, nch_max - 1), 1 - cur)
        c0 = wlo + CW * c
        c1 = jnp.minimum(c0 + CW, nwf)
        eff = jnp.minimum(c0, nwf - CW) * 128
        ngc = _compact(mid_v, mpos_v, cmc_v, cmt_v, ngroups, c0, c1, eff)
        return _extract(cmc_v, cmt_v, ngc, wbuf_v.at[cur], rows_h, pos_h,
                        rbase, pbase, stage_d, stage_p, ssem, st, False)

    state = lax.fori_loop(0, nch_max, chunk_body, state)
    pltpu.make_async_copy(tab_h.at[:, pl.ds(0, CU)],
                          wbuf_v.at[nch_max & 1], dsem).wait()

    ngc = _compact(mid_v, mpos_v, cmc_v, cmt_v, ngroups, nwf, nwf + 1, tail0)
    return _extract(cmc_v, cmt_v, ngc, tail_v, rows_h, pos_h, rbase, pbase,
                    stage_d, stage_p, ssem, state, True)


@functools.partial(
    pl.kernel,
    out_type=[
        jax.ShapeDtypeStruct((NTILES * B * F,), jnp.float32),
        jax.ShapeDtypeStruct((NTILES * B,), jnp.int32),
        jax.ShapeDtypeStruct((NTILES * B * F,), jnp.float32),
        jax.ShapeDtypeStruct((NTILES * B,), jnp.int32),
        jax.ShapeDtypeStruct((2 * NTILES * L,), jnp.int32),
    ],
    mesh=_MESH,
    compiler_params=pltpu.CompilerParams(needs_layout_passes=False),
    scratch_types=[
        pltpu.VMEM((B,), jnp.int32),
        pltpu.VMEM((B + L,), jnp.int32),
        pltpu.VMEM((B + L,), jnp.int32),
        pltpu.VMEM((B + L,), jnp.int32),
        pltpu.VMEM((B + L,), jnp.int32),
        pltpu.VMEM((2, F, CU), jnp.float32),
        pltpu.VMEM((TAIL_U, 128), jnp.float32),
        pltpu.VMEM((512,), jnp.float32),
        pltpu.VMEM((2 * L,), jnp.int32),
        pltpu.VMEM((2 * L,), jnp.int32),
        pltpu.SemaphoreType.DMA,
        pltpu.SemaphoreType.DMA,
    ],
)
def _sc_gather(user_h, movie_h, ut_h, mt_h, tailu_h, tailm_h,
               rows_u_h, tgt_u_h, rows_m_h, tgt_m_h, cnt_h,
               ids_v, mid_v, mpos_v, cmc_v, cmt_v, wbuf_v, tail_v,
               stage_d, stage_p, cnt_v, dsem, ssem):
    wid = lax.axis_index("s") * 2 + lax.axis_index("c")
    pltpu.sync_copy(tailu_h, tail_v)
    st = _phase(ut_h, user_h, rows_u_h, tgt_u_h, NWF_U, NCH_U, TAIL_U0,
                tail_v, ids_v, mid_v, mpos_v, cmc_v, cmt_v, wbuf_v,
                stage_d, stage_p, dsem, ssem, wid, (0, 0))
    n_u = st[1]
    pltpu.sync_copy(tailm_h, tail_v.at[pl.ds(0, TAIL_M)])
    st = _phase(mt_h, movie_h, rows_m_h, tgt_m_h, NWF_M, NCH_M, TAIL_M0,
                tail_v, ids_v, mid_v, mpos_v, cmc_v, cmt_v, wbuf_v,
                stage_d, stage_p, dsem, ssem, wid, (st[0], 0))
    n_m = st[1]

    def drain(j, c):
        pltpu.make_async_copy(stage_d.at[pl.ds((j & 1) * 256, 256)],
                              rows_u_h.at[pl.ds(0, 256)], ssem).wait()
        pltpu.make_async_copy(stage_p.at[pl.ds((j & 1) * L, L)],
                              tgt_u_h.at[pl.ds(0, L)], ssem).wait()
        return c

    lax.fori_loop(0, st[0], drain, 0)

    cnt_v[pl.ds(0, L)] = jnp.full((L,), 1, jnp.int32) * n_u
    cnt_v[pl.ds(L, L)] = jnp.full((L,), 1, jnp.int32) * n_m
    pltpu.sync_copy(cnt_v.at[pl.ds(0, L)], cnt_h.at[pl.ds(wid * L, L)])
    pltpu.sync_copy(cnt_v.at[pl.ds(L, L)],
                    cnt_h.at[pl.ds(NTILES * L + wid * L, L)])


def _scatter_one(rows_h, tgt_h, out_h, n_groups, rbase, pbase,
                 rbuf, pbuf, dsem, ssem):
    def wait_one(slot):
        pltpu.make_async_copy(rbuf.at[slot],
                              out_h.at[pbuf.at[pl.ds(slot * L, L)]],
                              ssem).wait()

    def ext(g):
        slot = g & 1
        pltpu.sync_copy(rows_h.at[pl.ds(rbase + g * L, L)], rbuf.at[slot])
        pltpu.sync_copy(tgt_h.at[pl.ds(pbase + g * L, L)],
                        pbuf.at[pl.ds(slot * L, L)])
        sl = pl.ds(slot * L, L)
        pbuf[sl] = pbuf[sl] >> 4
        pltpu.async_copy(rbuf.at[slot], out_h.at[pbuf.at[sl]], ssem)

    lim = jnp.minimum(n_groups, 2)

    def abody(g, c):
        ext(g)
        return c

    def bbody(g, c):
        wait_one(g & 1)
        ext(g)
        return c

    def dbody(j, c):
        wait_one(j & 1)
        return c

    lax.fori_loop(0, lim, abody, 0)
    lax.fori_loop(lim, n_groups, bbody, 0)
    lax.fori_loop(0, lim, dbody, 0)


@functools.partial(
    pl.kernel,
    out_type=[
        jax.ShapeDtypeStruct((B + L, F), jnp.float32),
        jax.ShapeDtypeStruct((B + L, F), jnp.float32),
    ],
    mesh=_MESH,
    compiler_params=pltpu.CompilerParams(needs_layout_passes=False,
                                         use_tc_tiling_on_sc=False),
    scratch_types=[
        pltpu.VMEM((2, L, F), jnp.float32),
        pltpu.VMEM((2 * L,), jnp.int32),
        pltpu.VMEM((2 * L,), jnp.int32),
        pltpu.SemaphoreType.DMA,
        pltpu.SemaphoreType.DMA,
    ],
)
def _sc_scatter(rows_u_h, tgt_u_h, rows_m_h, tgt_m_h, cnt_h, uo_h, mo_h,
                rbuf, pbuf, cnt_v, dsem, ssem):
    wid = lax.axis_index("s") * 2 + lax.axis_index("c")
    pltpu.sync_copy(cnt_h.at[pl.ds(wid * L, L)], cnt_v.at[pl.ds(0, L)])
    pltpu.sync_copy(cnt_h.at[pl.ds(NTILES * L + wid * L, L)],
                    cnt_v.at[pl.ds(L, L)])
    n_u = jnp.max(cnt_v[pl.ds(0, L)])
    n_m = jnp.max(cnt_v[pl.ds(L, L)])
    _scatter_one(rows_u_h, tgt_u_h, uo_h, n_u, wid * B, wid * B,
                 rbuf, pbuf, dsem, ssem)
    _scatter_one(rows_m_h, tgt_m_h, mo_h, n_m, wid * B, wid * B,
                 rbuf, pbuf, dsem, ssem)


def _mlp_body(u_ref, m_ref, w1u_ref, w1m_ref, b1_ref, w2_ref, b2_ref, o_ref):
    h = jnp.dot(u_ref[...], w1u_ref[...], preferred_element_type=jnp.float32)
    h = h + jnp.dot(m_ref[...], w1m_ref[...], preferred_element_type=jnp.float32)
    h = jnp.maximum(h + b1_ref[...], 0.0)
    o = jnp.dot(h, w2_ref[...], preferred_element_type=jnp.float32) + b2_ref[...]
    o_ref[...] = 5.5 / (1.0 + jnp.exp(-o))


def _mlp(u_emb, m_emb, w1u, w1m, b1, w2, b2):
    return pl.pallas_call(
        _mlp_body,
        out_shape=jax.ShapeDtypeStruct((B, 1), jnp.float32),
    )(u_emb, m_emb, w1u, w1m, b1[None], w2, b2[None])


def kernel(user, movie, u_table, m_table, W1, b1, W2, b2):
    user = user.astype(jnp.int32)
    movie = movie.astype(jnp.int32)
    pad = ((0, 0), (0, 128 - F))
    tailu = jnp.pad(u_table[TAIL_U0:], pad)
    tailm = jnp.pad(m_table[TAIL_M0:], pad)
    rows_u, tgt_u, rows_m, tgt_m, cnt = _sc_gather(
        user, movie, u_table.T, m_table.T, tailu, tailm)
    uo, mo = _sc_scatter(rows_u.reshape(NTILES * B, F), tgt_u,
                         rows_m.reshape(NTILES * B, F), tgt_m, cnt)
    return _mlp(uo[:B], mo[:B], W1[:F], W1[F:], b1, W2, b2)


# speedup vs baseline: 97.3220x; 1.1784x over previous
import functools

import jax
import jax.numpy as jnp
from jax import lax
from jax.experimental import pallas as pl
from jax.experimental.pallas import tpu as pltpu
from jax.experimental.pallas import tpu_sc as plsc

B = 16384
F = 16
NU = 1000000
NM = 100000
L = 16
NTILES = 32
CW = 8
CU = CW * 128

NWF_U = NU // 128
NWF_M = NM // 128
TAIL_U0 = NWF_U * 128
TAIL_M0 = NWF_M * 128
TAIL_U = NU - TAIL_U0
TAIL_M = NM - TAIL_M0
NCH_U = 31
NCH_M = 4
SAFE = B * F

_MESH = plsc.VectorSubcoreMesh(core_axis_name="c", subcore_axis_name="s")


def _wrange(wid, nwf):
    per = nwf // NTILES
    rem = nwf - per * NTILES
    wlo = wid * per + jnp.minimum(wid, rem)
    cnt = per + (wid < rem).astype(jnp.int32)
    whi = wlo + cnt + (wid == NTILES - 1).astype(jnp.int32)
    return wlo, whi


def _scan(ids_v, mid_v, mpos_v, wlo, whi):
    def body(g, n):
        idv = ids_v[pl.ds(g * L, L)]
        w = idv >> 7
        msk = (w >= wlo) & (w < whi)
        posv = g * L + lax.iota(jnp.int32, L)
        plsc.store_compressed(mid_v.at[pl.ds(n, L)], idv, mask=msk)
        plsc.store_compressed(mpos_v.at[pl.ds(n, L)], posv, mask=msk)
        return n + jnp.sum(msk.astype(jnp.int32))

    n = lax.fori_loop(0, B // L, body, 0)
    mid_v[pl.ds(n, L)] = jnp.full((L,), -1, jnp.int32)
    return (n + L - 1) >> 4


def _compact(mid_v, mpos_v, cmc_v, cmt_v, ngroups, lo_w, hi_w, col_base):
    def body(g, nc):
        idv = mid_v[pl.ds(g * L, L)]
        posv = mpos_v[pl.ds(g * L, L)]
        w = idv >> 7
        msk = (w >= lo_w) & (w < hi_w)
        plsc.store_compressed(cmc_v.at[pl.ds(nc, L)], idv - col_base, mask=msk)
        plsc.store_compressed(cmt_v.at[pl.ds(nc, L)], posv * F, mask=msk)
        return nc + jnp.sum(msk.astype(jnp.int32))

    nc = lax.fori_loop(0, ngroups, body, 0)
    cmc_v[pl.ds(nc, L)] = jnp.full((L,), 0, jnp.int32)
    cmt_v[pl.ds(nc, L)] = SAFE + lax.iota(jnp.int32, L) * F
    return (nc + L - 1) >> 4


def _extract(cmc_v, cmt_v, ngc, src_v, rows_h, pos_h, rbase, pbase,
             stage_d, stage_p, ssem, state, row_is_id):
    prev_out, k0 = state

    def wait_trio(slot):
        pltpu.make_async_copy(stage_d.at[pl.ds(slot * 256, 256)],
                              rows_h.at[pl.ds(0, 256)], ssem).wait()
        pltpu.make_async_copy(stage_p.at[pl.ds(slot * L, L)],
                              pos_h.at[pl.ds(0, L)], ssem).wait()

    def drain(j, c):
        wait_trio(j & 1)
        return c

    lax.fori_loop(0, prev_out, drain, 0)

    def ext(g):
        slot = (g - k0) & 1
        colv = cmc_v[pl.ds((g - k0) * L, L)]
        tgtv = cmt_v[pl.ds((g - k0) * L, L)]
        iota = lax.iota(jnp.int32, L)
        for f in range(F):
            fv = jnp.full((L,), f, jnp.int32)
            if row_is_id:
                vals = plsc.load_gather(src_v, [colv, fv])
            else:
                vals = plsc.load_gather(src_v, [fv, colv])
            plsc.store_scatter(stage_d, [slot * 256 + iota * F + f], vals)
        stage_p[pl.ds(slot * L, L)] = tgtv
        pltpu.async_copy(stage_d.at[pl.ds(slot * 256, 256)],
                         rows_h.at[pl.ds(rbase + g * 256, 256)], ssem)
        pltpu.async_copy(stage_p.at[pl.ds(slot * L, L)],
                         pos_h.at[pl.ds(pbase + g * L, L)], ssem)

    lim = jnp.minimum(ngc, 2)

    def abody(g, c):
        ext(g)
        return c

    def bbody(g, c):
        wait_trio((g - k0) & 1)
        ext(g)
        return c

    lax.fori_loop(k0, k0 + lim, abody, 0)
    lax.fori_loop(k0 + lim, k0 + ngc, bbody, 0)
    return (lim, k0 + ngc)


def _phase(tab_h, ids_h, rows_h, pos_h, nwf, nch_max, tail0, tail_v,
           ids_v, mid_v, mpos_v, cmc_v, cmt_v, wbuf_v, stage_d, stage_p,
           dsem, ssem, wid, state):
    pltpu.sync_copy(ids_h, ids_v)
    wlo, whi = _wrange(wid, nwf)
    rbase = wid * (B * F)
    pbase = wid * B

    def fire(ci, slot):
        eff = pl.multiple_of(
            jnp.minimum(wlo + CW * ci, nwf - CW) * 128, 128)
        pltpu.async_copy(tab_h.at[:, pl.ds(eff, CU)], wbuf_v.at[slot], dsem)

    fire(0, 0)
    ngroups = _scan(ids_v, mid_v, mpos_v, wlo, whi)

    def chunk_body(c, st):
        cur = c & 1
        pltpu.make_async_copy(tab_h.at[:, pl.ds(0, CU)], wbuf_v.at[cur],
                              dsem).wait()
        fire(jnp.minimum(c + 1, nch_max - 1), 1 - cur)
        c0 = wlo + CW * c
        c1 = jnp.minimum(c0 + CW, nwf)
        eff = jnp.minimum(c0, nwf - CW) * 128
        ngc = _compact(mid_v, mpos_v, cmc_v, cmt_v, ngroups, c0, c1, eff)
        return _extract(cmc_v, cmt_v, ngc, wbuf_v.at[cur], rows_h, pos_h,
                        rbase, pbase, stage_d, stage_p, ssem, st, False)

    state = lax.fori_loop(0, nch_max, chunk_body, state)
    pltpu.make_async_copy(tab_h.at[:, pl.ds(0, CU)],
                          wbuf_v.at[nch_max & 1], dsem).wait()

    ngc = _compact(mid_v, mpos_v, cmc_v, cmt_v, ngroups, nwf, nwf + 1, tail0)
    return _extract(cmc_v, cmt_v, ngc, tail_v, rows_h, pos_h, rbase, pbase,
                    stage_d, stage_p, ssem, state, True)


@functools.partial(
    pl.kernel,
    out_type=[
        jax.ShapeDtypeStruct((NTILES * B * F,), jnp.float32),
        jax.ShapeDtypeStruct((NTILES * B,), jnp.int32),
        jax.ShapeDtypeStruct((NTILES * B * F,), jnp.float32),
        jax.ShapeDtypeStruct((NTILES * B,), jnp.int32),
        jax.ShapeDtypeStruct((2 * NTILES * L,), jnp.int32),
    ],
    mesh=_MESH,
    compiler_params=pltpu.CompilerParams(needs_layout_passes=False),
    scratch_types=[
        pltpu.VMEM((B,), jnp.int32),
        pltpu.VMEM((B + L,), jnp.int32),
        pltpu.VMEM((B + L,), jnp.int32),
        pltpu.VMEM((B + L,), jnp.int32),
        pltpu.VMEM((B + L,), jnp.int32),
        pltpu.VMEM((2, F, CU), jnp.float32),
        pltpu.VMEM((TAIL_U, 128), jnp.float32),
        pltpu.VMEM((512,), jnp.float32),
        pltpu.VMEM((2 * L,), jnp.int32),
        pltpu.VMEM((2 * L,), jnp.int32),
        pltpu.SemaphoreType.DMA,
        pltpu.SemaphoreType.DMA,
    ],
)
def _sc_gather(user_h, movie_h, ut_h, mt_h, tailu_h, tailm_h,
               rows_u_h, tgt_u_h, rows_m_h, tgt_m_h, cnt_h,
               ids_v, mid_v, mpos_v, cmc_v, cmt_v, wbuf_v, tail_v,
               stage_d, stage_p, cnt_v, dsem, ssem):
    wid = lax.axis_index("s") * 2 + lax.axis_index("c")
    pltpu.sync_copy(tailu_h, tail_v)
    st = _phase(ut_h, user_h, rows_u_h, tgt_u_h, NWF_U, NCH_U, TAIL_U0,
                tail_v, ids_v, mid_v, mpos_v, cmc_v, cmt_v, wbuf_v,
                stage_d, stage_p, dsem, ssem, wid, (0, 0))
    n_u = st[1]
    pltpu.sync_copy(tailm_h, tail_v.at[pl.ds(0, TAIL_M)])
    st = _phase(mt_h, movie_h, rows_m_h, tgt_m_h, NWF_M, NCH_M, TAIL_M0,
                tail_v, ids_v, mid_v, mpos_v, cmc_v, cmt_v, wbuf_v,
                stage_d, stage_p, dsem, ssem, wid, (st[0], 0))
    n_m = st[1]

    def drain(j, c):
        pltpu.make_async_copy(stage_d.at[pl.ds((j & 1) * 256, 256)],
                              rows_u_h.at[pl.ds(0, 256)], ssem).wait()
        pltpu.make_async_copy(stage_p.at[pl.ds((j & 1) * L, L)],
                              tgt_u_h.at[pl.ds(0, L)], ssem).wait()
        return c

    lax.fori_loop(0, st[0], drain, 0)

    cnt_v[pl.ds(0, L)] = jnp.full((L,), 1, jnp.int32) * n_u
    cnt_v[pl.ds(L, L)] = jnp.full((L,), 1, jnp.int32) * n_m
    pltpu.sync_copy(cnt_v.at[pl.ds(0, L)], cnt_h.at[pl.ds(wid * L, L)])
    pltpu.sync_copy(cnt_v.at[pl.ds(L, L)],
                    cnt_h.at[pl.ds(NTILES * L + wid * L, L)])


CG = 2048


def _scatter_one(rows_h, tgt_h, out_h, n_groups, rbase, pbase,
                 rbuf, pbuf, dsem, ssem):
    def cbody(c, _):
        pltpu.sync_copy(rows_h.at[pl.ds(rbase + c * CG, CG)], rbuf)
        pltpu.sync_copy(tgt_h.at[pl.ds(pbase + c * CG, CG)],
                        pbuf.at[pl.ds(0, CG)])

        def sbody(i, _):
            sl = pl.ds(i * L, L)
            pbuf[sl] = pbuf[sl] >> 4
            return 0

        lax.fori_loop(0, CG // L, sbody, 0)
        rem = jnp.minimum(n_groups - c * (CG // L), CG // L)
        for i in range(7):
            pbuf[pl.ds(rem * L + i * L, L)] = jnp.full((L,), B, jnp.int32)
        nsc = (rem * L + 127) >> 7

        def fire(j, _):
            pltpu.async_copy(rbuf.at[pl.ds(j * 128, 128)],
                             out_h.at[pbuf.at[pl.ds(j * 128, 128)]], ssem)
            return 0

        def wait(j, _):
            pltpu.make_async_copy(rbuf.at[pl.ds(0, 128)],
                                  out_h.at[pbuf.at[pl.ds(0, 128)]],
                                  ssem).wait()
            return 0

        lax.fori_loop(0, nsc, fire, 0)
        lax.fori_loop(0, nsc, wait, 0)
        return 0

    nch = (n_groups + CG // L - 1) // (CG // L)
    lax.fori_loop(0, nch, cbody, 0)


@functools.partial(
    pl.kernel,
    out_type=[
        jax.ShapeDtypeStruct((B + L, F), jnp.float32),
        jax.ShapeDtypeStruct((B + L, F), jnp.float32),
    ],
    mesh=_MESH,
    compiler_params=pltpu.CompilerParams(needs_layout_passes=False,
                                         use_tc_tiling_on_sc=False),
    scratch_types=[
        pltpu.VMEM((CG, F), jnp.float32),
        pltpu.VMEM((CG + 128,), jnp.int32),
        pltpu.VMEM((2 * L,), jnp.int32),
        pltpu.SemaphoreType.DMA,
        pltpu.SemaphoreType.DMA,
    ],
)
def _sc_scatter(rows_u_h, tgt_u_h, rows_m_h, tgt_m_h, cnt_h, uo_h, mo_h,
                rbuf, pbuf, cnt_v, dsem, ssem):
    wid = lax.axis_index("s") * 2 + lax.axis_index("c")
    pltpu.sync_copy(cnt_h.at[pl.ds(wid * L, L)], cnt_v.at[pl.ds(0, L)])
    pltpu.sync_copy(cnt_h.at[pl.ds(NTILES * L + wid * L, L)],
                    cnt_v.at[pl.ds(L, L)])
    n_u = jnp.max(cnt_v[pl.ds(0, L)])
    n_m = jnp.max(cnt_v[pl.ds(L, L)])
    _scatter_one(rows_u_h, tgt_u_h, uo_h, n_u, wid * B, wid * B,
                 rbuf, pbuf, dsem, ssem)
    _scatter_one(rows_m_h, tgt_m_h, mo_h, n_m, wid * B, wid * B,
                 rbuf, pbuf, dsem, ssem)


def _mlp_body(u_ref, m_ref, w1u_ref, w1m_ref, b1_ref, w2_ref, b2_ref, o_ref):
    up = u_ref[...].reshape(B * F // 128, 128)
    mp = m_ref[...].reshape(B * F // 128, 128)
    h = jnp.dot(up, w1u_ref[...], preferred_element_type=jnp.float32)
    h = h + jnp.dot(mp, w1m_ref[...], preferred_element_type=jnp.float32)
    h = jnp.maximum(h + b1_ref[...], 0.0)
    o = jnp.dot(h, w2_ref[...], preferred_element_type=jnp.float32) + b2_ref[...]
    o_ref[...] = 5.5 / (1.0 + jnp.exp(-o))


def _mlp(u_flat, m_flat, w1u, w1m, b1, w2, b2):
    eye = jnp.eye(8, dtype=jnp.float32)
    return pl.pallas_call(
        _mlp_body,
        out_shape=jax.ShapeDtypeStruct((B * F // 128, 8), jnp.float32),
    )(u_flat, m_flat, jnp.kron(eye, w1u), jnp.kron(eye, w1m),
      jnp.tile(b1, 8)[None], jnp.kron(eye, w2), jnp.tile(b2, 8)[None])


def kernel(user, movie, u_table, m_table, W1, b1, W2, b2):
    user = user.astype(jnp.int32)
    movie = movie.astype(jnp.int32)
    pad = ((0, 0), (0, 128 - F))
    tailu = jnp.pad(u_table[TAIL_U0:], pad)
    tailm = jnp.pad(m_table[TAIL_M0:], pad)
    rows_u, tgt_u, rows_m, tgt_m, cnt = _sc_gather(
        user, movie, u_table.T, m_table.T, tailu, tailm)
    uo, mo = _sc_scatter(rows_u.reshape(NTILES * B, F), tgt_u,
                         rows_m.reshape(NTILES * B, F), tgt_m, cnt)
    out = _mlp(uo[:B].reshape(B * F), mo[:B].reshape(B * F),
               W1[:F], W1[F:], b1, W2, b2)
    return out.reshape(B, 1)


# speedup vs baseline: 106.3604x; 1.0929x over previous
import functools

import jax
import jax.numpy as jnp
from jax import lax
from jax.experimental import pallas as pl
from jax.experimental.pallas import tpu as pltpu
from jax.experimental.pallas import tpu_sc as plsc

B = 16384
F = 16
NU = 1000000
NM = 100000
L = 16
NTILES = 32
CW = 8
CU = CW * 128

NWF_U = NU // 128
NWF_M = NM // 128
TAIL_U0 = NWF_U * 128
TAIL_M0 = NWF_M * 128
TAIL_U = NU - TAIL_U0
TAIL_M = NM - TAIL_M0
NCH_U = 31
NCH_M = 4
SAFE = B * F

_MESH = plsc.VectorSubcoreMesh(core_axis_name="c", subcore_axis_name="s")


def _wrange(wid, nwf):
    per = nwf // NTILES
    rem = nwf - per * NTILES
    wlo = wid * per + jnp.minimum(wid, rem)
    cnt = per + (wid < rem).astype(jnp.int32)
    whi = wlo + cnt + (wid == NTILES - 1).astype(jnp.int32)
    return wlo, whi


def _scan(ids_v, mid_v, mpos_v, wlo, whi):
    def body(g, n):
        idv = ids_v[pl.ds(g * L, L)]
        w = idv >> 7
        msk = (w >= wlo) & (w < whi)
        posv = g * L + lax.iota(jnp.int32, L)
        plsc.store_compressed(mid_v.at[pl.ds(n, L)], idv, mask=msk)
        plsc.store_compressed(mpos_v.at[pl.ds(n, L)], posv, mask=msk)
        return n + jnp.sum(msk.astype(jnp.int32))

    n = lax.fori_loop(0, B // L, body, 0)
    mid_v[pl.ds(n, L)] = jnp.full((L,), -1, jnp.int32)
    return (n + L - 1) >> 4


def _compact(mid_v, mpos_v, cmc_v, cmt_v, ngroups, lo_w, hi_w, col_base):
    def body(g, nc):
        idv = mid_v[pl.ds(g * L, L)]
        posv = mpos_v[pl.ds(g * L, L)]
        w = idv >> 7
        msk = (w >= lo_w) & (w < hi_w)
        plsc.store_compressed(cmc_v.at[pl.ds(nc, L)], idv - col_base, mask=msk)
        plsc.store_compressed(cmt_v.at[pl.ds(nc, L)], posv * F, mask=msk)
        return nc + jnp.sum(msk.astype(jnp.int32))

    nc = lax.fori_loop(0, ngroups, body, 0)
    cmc_v[pl.ds(nc, L)] = jnp.full((L,), 0, jnp.int32)
    cmt_v[pl.ds(nc, L)] = SAFE + lax.iota(jnp.int32, L) * F
    return (nc + L - 1) >> 4


def _extract(cmc_v, cmt_v, ngc, src_v, rows_h, pos_h, rbase, pbase,
             stage_d, stage_p, ssem, state, row_is_id):
    prev_out, k0 = state

    def wait_trio(slot):
        pltpu.make_async_copy(stage_d.at[pl.ds(slot * 256, 256)],
                              rows_h.at[pl.ds(0, 256)], ssem).wait()
        pltpu.make_async_copy(stage_p.at[pl.ds(slot * L, L)],
                              pos_h.at[pl.ds(0, L)], ssem).wait()

    def drain(j, c):
        wait_trio(j & 1)
        return c

    lax.fori_loop(0, prev_out, drain, 0)

    def ext(g):
        slot = (g - k0) & 1
        colv = cmc_v[pl.ds((g - k0) * L, L)]
        tgtv = cmt_v[pl.ds((g - k0) * L, L)]
        iota = lax.iota(jnp.int32, L)
        for f in range(F):
            fv = jnp.full((L,), f, jnp.int32)
            if row_is_id:
                vals = plsc.load_gather(src_v, [colv, fv])
            else:
                vals = plsc.load_gather(src_v, [fv, colv])
            plsc.store_scatter(stage_d, [slot * 256 + iota * F + f], vals)
        stage_p[pl.ds(slot * L, L)] = tgtv
        pltpu.async_copy(stage_d.at[pl.ds(slot * 256, 256)],
                         rows_h.at[pl.ds(rbase + g * 256, 256)], ssem)
        pltpu.async_copy(stage_p.at[pl.ds(slot * L, L)],
                         pos_h.at[pl.ds(pbase + g * L, L)], ssem)

    lim = jnp.minimum(ngc, 2)

    def abody(g, c):
        ext(g)
        return c

    def bbody(g, c):
        wait_trio((g - k0) & 1)
        ext(g)
        return c

    lax.fori_loop(k0, k0 + lim, abody, 0)
    lax.fori_loop(k0 + lim, k0 + ngc, bbody, 0)
    return (lim, k0 + ngc)


def _phase(tab_h, ids_h, rows_h, pos_h, nwf, nch_max, tail0, tail_v,
           ids_v, mid_v, mpos_v, cmc_v, cmt_v, wbuf_v, stage_d, stage_p,
           dsem, ssem, wid, state):
    pltpu.sync_copy(ids_h, ids_v)
    wlo, whi = _wrange(wid, nwf)
    rbase = wid * (B * F)
    pbase = wid * B

    def fire(ci, slot):
        eff = pl.multiple_of(
            jnp.minimum(wlo + CW * ci, nwf - CW) * 128, 128)
        pltpu.async_copy(tab_h.at[:, pl.ds(eff, CU)], wbuf_v.at[slot], dsem)

    fire(0, 0)
    ngroups = _scan(ids_v, mid_v, mpos_v, wlo, whi)

    def chunk_body(c, st):
        cur = c & 1
        pltpu.make_async_copy(tab_h.at[:, pl.ds(0, CU)], wbuf_v.at[cur],
                              dsem).wait()
        fire(jnp.minimum(c + 1, nch_max - 1), 1 - cur)
        c0 = wlo + CW * c
        c1 = jnp.minimum(c0 + CW, nwf)
        eff = jnp.minimum(c0, nwf - CW) * 128
        ngc = _compact(mid_v, mpos_v, cmc_v, cmt_v, ngroups, c0, c1, eff)
        return _extract(cmc_v, cmt_v, ngc, wbuf_v.at[cur], rows_h, pos_h,
                        rbase, pbase, stage_d, stage_p, ssem, st, False)

    state = lax.fori_loop(0, nch_max, chunk_body, state)
    pltpu.make_async_copy(tab_h.at[:, pl.ds(0, CU)],
                          wbuf_v.at[nch_max & 1], dsem).wait()

    ngc = _compact(mid_v, mpos_v, cmc_v, cmt_v, ngroups, nwf, nwf + 1, tail0)
    return _extract(cmc_v, cmt_v, ngc, tail_v, rows_h, pos_h, rbase, pbase,
                    stage_d, stage_p, ssem, state, True)


@functools.partial(
    pl.kernel,
    out_type=[
        jax.ShapeDtypeStruct((NTILES * B * F,), jnp.float32),
        jax.ShapeDtypeStruct((NTILES * B,), jnp.int32),
        jax.ShapeDtypeStruct((NTILES * B * F,), jnp.float32),
        jax.ShapeDtypeStruct((NTILES * B,), jnp.int32),
        jax.ShapeDtypeStruct((2 * NTILES * L,), jnp.int32),
    ],
    mesh=_MESH,
    compiler_params=pltpu.CompilerParams(needs_layout_passes=False),
    scratch_types=[
        pltpu.VMEM((B,), jnp.int32),
        pltpu.VMEM((B + L,), jnp.int32),
        pltpu.VMEM((B + L,), jnp.int32),
        pltpu.VMEM((B + L,), jnp.int32),
        pltpu.VMEM((B + L,), jnp.int32),
        pltpu.VMEM((2, F, CU), jnp.float32),
        pltpu.VMEM((TAIL_U, 128), jnp.float32),
        pltpu.VMEM((512,), jnp.float32),
        pltpu.VMEM((2 * L,), jnp.int32),
        pltpu.VMEM((2 * L,), jnp.int32),
        pltpu.SemaphoreType.DMA,
        pltpu.SemaphoreType.DMA,
    ],
)
def _sc_gather(user_h, movie_h, ut_h, mt_h, tailu_h, tailm_h,
               rows_u_h, tgt_u_h, rows_m_h, tgt_m_h, cnt_h,
               ids_v, mid_v, mpos_v, cmc_v, cmt_v, wbuf_v, tail_v,
               stage_d, stage_p, cnt_v, dsem, ssem):
    wid = lax.axis_index("s") * 2 + lax.axis_index("c")
    pltpu.sync_copy(tailu_h, tail_v)
    st = _phase(ut_h, user_h, rows_u_h, tgt_u_h, NWF_U, NCH_U, TAIL_U0,
                tail_v, ids_v, mid_v, mpos_v, cmc_v, cmt_v, wbuf_v,
                stage_d, stage_p, dsem, ssem, wid, (0, 0))
    n_u = st[1]
    pltpu.sync_copy(tailm_h, tail_v.at[pl.ds(0, TAIL_M)])
    st = _phase(mt_h, movie_h, rows_m_h, tgt_m_h, NWF_M, NCH_M, TAIL_M0,
                tail_v, ids_v, mid_v, mpos_v, cmc_v, cmt_v, wbuf_v,
                stage_d, stage_p, dsem, ssem, wid, (st[0], 0))
    n_m = st[1]

    def drain(j, c):
        pltpu.make_async_copy(stage_d.at[pl.ds((j & 1) * 256, 256)],
                              rows_u_h.at[pl.ds(0, 256)], ssem).wait()
        pltpu.make_async_copy(stage_p.at[pl.ds((j & 1) * L, L)],
                              tgt_u_h.at[pl.ds(0, L)], ssem).wait()
        return c

    lax.fori_loop(0, st[0], drain, 0)

    cnt_v[pl.ds(0, L)] = jnp.full((L,), 1, jnp.int32) * n_u
    cnt_v[pl.ds(L, L)] = jnp.full((L,), 1, jnp.int32) * n_m
    pltpu.sync_copy(cnt_v.at[pl.ds(0, L)], cnt_h.at[pl.ds(wid * L, L)])
    pltpu.sync_copy(cnt_v.at[pl.ds(L, L)],
                    cnt_h.at[pl.ds(NTILES * L + wid * L, L)])


CG = 512


def _scatter_one(rows_h, tgt_h, out_h, n_groups, rbase, pbase,
                 rbuf, pbuf, dsem, ssem):
    def cbody(c, _):
        pltpu.sync_copy(rows_h.at[pl.ds(rbase + c * CG, CG)], rbuf)
        pltpu.sync_copy(tgt_h.at[pl.ds(pbase + c * CG, CG)],
                        pbuf.at[pl.ds(0, CG)])

        def sbody(i, _):
            sl = pl.ds(i * L, L)
            pbuf[sl] = pbuf[sl] >> 4
            return 0

        lax.fori_loop(0, CG // L, sbody, 0)
        rem = jnp.minimum(n_groups - c * (CG // L), CG // L)
        for i in range(7):
            pbuf[pl.ds(rem * L + i * L, L)] = jnp.full((L,), B, jnp.int32)
        nsc = (rem * L + 127) >> 7

        def fire(j, _):
            pltpu.async_copy(rbuf.at[pl.ds(j * 128, 128)],
                             out_h.at[pbuf.at[pl.ds(j * 128, 128)]], ssem)
            return 0

        def wait(j, _):
            pltpu.make_async_copy(rbuf.at[pl.ds(0, 128)],
                                  out_h.at[pbuf.at[pl.ds(0, 128)]],
                                  ssem).wait()
            return 0

        lax.fori_loop(0, nsc, fire, 0)
        lax.fori_loop(0, nsc, wait, 0)
        return 0

    nch = (n_groups + CG // L - 1) // (CG // L)
    lax.fori_loop(0, nch, cbody, 0)


@functools.partial(
    pl.kernel,
    out_type=[
        jax.ShapeDtypeStruct((B + L, F), jnp.float32),
        jax.ShapeDtypeStruct((B + L, F), jnp.float32),
    ],
    mesh=_MESH,
    compiler_params=pltpu.CompilerParams(needs_layout_passes=False,
                                         use_tc_tiling_on_sc=False),
    scratch_types=[
        pltpu.VMEM((CG, F), jnp.float32),
        pltpu.VMEM((CG + 128,), jnp.int32),
        pltpu.VMEM((2 * L,), jnp.int32),
        pltpu.SemaphoreType.DMA,
        pltpu.SemaphoreType.DMA,
    ],
)
def _sc_scatter(rows_u_h, tgt_u_h, rows_m_h, tgt_m_h, cnt_h, uo_h, mo_h,
                rbuf, pbuf, cnt_v, dsem, ssem):
    wid = lax.axis_index("s") * 2 + lax.axis_index("c")
    pltpu.sync_copy(cnt_h.at[pl.ds(wid * L, L)], cnt_v.at[pl.ds(0, L)])
    pltpu.sync_copy(cnt_h.at[pl.ds(NTILES * L + wid * L, L)],
                    cnt_v.at[pl.ds(L, L)])
    n_u = jnp.max(cnt_v[pl.ds(0, L)])
    n_m = jnp.max(cnt_v[pl.ds(L, L)])
    _scatter_one(rows_u_h, tgt_u_h, uo_h, n_u, wid * B, wid * B,
                 rbuf, pbuf, dsem, ssem)
    _scatter_one(rows_m_h, tgt_m_h, mo_h, n_m, wid * B, wid * B,
                 rbuf, pbuf, dsem, ssem)


def _mlp_body(u_ref, m_ref, w1u_ref, w1m_ref, b1_ref, w2_ref, b2_ref, o_ref):
    h = jnp.dot(u_ref[...], w1u_ref[...], preferred_element_type=jnp.float32)
    h = h + jnp.dot(m_ref[...], w1m_ref[...], preferred_element_type=jnp.float32)
    h = jnp.maximum(h + b1_ref[...], 0.0)
    o = jnp.dot(h, w2_ref[...], preferred_element_type=jnp.float32) + b2_ref[...]
    o_ref[...] = 5.5 / (1.0 + jnp.exp(-o))


def _mlp(u_pad, m_pad, w1u, w1m, b1, w2, b2):
    emb_spec = pl.BlockSpec((B, F), lambda i: (0, 0))

    def full(shape):
        return pl.BlockSpec(shape, lambda i: (0, 0))

    return pl.pallas_call(
        _mlp_body,
        grid=(1,),
        in_specs=[emb_spec, emb_spec, full((F, 64)), full((F, 64)),
                  full((1, 64)), full((64, 1)), full((1, 1))],
        out_specs=full((B, 1)),
        out_shape=jax.ShapeDtypeStruct((B, 1), jnp.float32),
    )(u_pad, m_pad, w1u, w1m, b1[None], w2, b2[None])


def kernel(user, movie, u_table, m_table, W1, b1, W2, b2):
    user = user.astype(jnp.int32)
    movie = movie.astype(jnp.int32)
    pad = ((0, 0), (0, 128 - F))
    tailu = jnp.pad(u_table[TAIL_U0:], pad)
    tailm = jnp.pad(m_table[TAIL_M0:], pad)
    rows_u, tgt_u, rows_m, tgt_m, cnt = _sc_gather(
        user, movie, u_table.T, m_table.T, tailu, tailm)
    uo, mo = _sc_scatter(rows_u.reshape(NTILES * B, F), tgt_u,
                         rows_m.reshape(NTILES * B, F), tgt_m, cnt)
    return _mlp(uo, mo, W1[:F], W1[F:], b1, W2, b2)


# speedup vs baseline: 112.7781x; 1.0603x over previous
import functools

import jax
import jax.numpy as jnp
from jax import lax
from jax.experimental import pallas as pl
from jax.experimental.pallas import tpu as pltpu
from jax.experimental.pallas import tpu_sc as plsc

B = 16384
F = 16
NU = 1000000
NM = 100000
L = 16
NTILES = 32
CW = 8
CU = CW * 128

NWF_U = NU // 128
NWF_M = NM // 128
TAIL_U0 = NWF_U * 128
TAIL_M0 = NWF_M * 128
TAIL_U = NU - TAIL_U0
TAIL_M = NM - TAIL_M0
NCH_U = 31
NCH_M = 4
SAFE = B * F

_MESH = plsc.VectorSubcoreMesh(core_axis_name="c", subcore_axis_name="s")


def _wrange(wid, nwf):
    per = nwf // NTILES
    rem = nwf - per * NTILES
    wlo = wid * per + jnp.minimum(wid, rem)
    cnt = per + (wid < rem).astype(jnp.int32)
    whi = wlo + cnt + (wid == NTILES - 1).astype(jnp.int32)
    return wlo, whi


def _scan(ids_v, mid_v, mpos_v, wlo, whi):
    def body(q, n):
        iota = lax.iota(jnp.int32, L)
        parts = []
        for j in range(4):
            g = q * 4 + j
            idv = ids_v[pl.ds(g * L, L)]
            w = idv >> 7
            msk = (w >= wlo) & (w < whi)
            parts.append((idv, g * L + iota, msk,
                          jnp.sum(msk.astype(jnp.int32))))
        for idv, posv, msk, s in parts:
            plsc.store_compressed(mid_v.at[pl.ds(n, L)], idv, mask=msk)
            plsc.store_compressed(mpos_v.at[pl.ds(n, L)], posv, mask=msk)
            n = n + s
        return n

    n = lax.fori_loop(0, B // L // 4, body, 0)
    for j in range(4):
        mid_v[pl.ds(n + j * L, L)] = jnp.full((L,), -1, jnp.int32)
    return (n + L - 1) >> 4


def _compact(mid_v, mpos_v, cmc_v, cmt_v, ngroups, lo_w, hi_w, col_base):
    def body(q, nc):
        parts = []
        for j in range(4):
            g = q * 4 + j
            idv = mid_v[pl.ds(g * L, L)]
            posv = mpos_v[pl.ds(g * L, L)]
            w = idv >> 7
            msk = (w >= lo_w) & (w < hi_w)
            parts.append((idv, posv, msk, jnp.sum(msk.astype(jnp.int32))))
        for idv, posv, msk, s in parts:
            plsc.store_compressed(cmc_v.at[pl.ds(nc, L)], idv - col_base,
                                  mask=msk)
            plsc.store_compressed(cmt_v.at[pl.ds(nc, L)], posv * F, mask=msk)
            nc = nc + s
        return nc

    nc = lax.fori_loop(0, (ngroups + 3) >> 2, body, 0)
    cmc_v[pl.ds(nc, L)] = jnp.full((L,), 0, jnp.int32)
    cmt_v[pl.ds(nc, L)] = SAFE + lax.iota(jnp.int32, L) * F
    return (nc + L - 1) >> 4


def _extract(cmc_v, cmt_v, ngc, src_v, rows_h, pos_h, rbase, pbase,
             stage_d, stage_p, ssem, state, row_is_id):
    prev_out, k0 = state

    def wait_trio(slot):
        pltpu.make_async_copy(stage_d.at[pl.ds(slot * 256, 256)],
                              rows_h.at[pl.ds(0, 256)], ssem).wait()
        pltpu.make_async_copy(stage_p.at[pl.ds(slot * L, L)],
                              pos_h.at[pl.ds(0, L)], ssem).wait()

    def drain(j, c):
        wait_trio(j & 1)
        return c

    lax.fori_loop(0, prev_out, drain, 0)

    def ext(g):
        slot = (g - k0) & 1
        colv = cmc_v[pl.ds((g - k0) * L, L)]
        tgtv = cmt_v[pl.ds((g - k0) * L, L)]
        iota = lax.iota(jnp.int32, L)
        for f in range(F):
            fv = jnp.full((L,), f, jnp.int32)
            if row_is_id:
                vals = plsc.load_gather(src_v, [colv, fv])
            else:
                vals = plsc.load_gather(src_v, [fv, colv])
            plsc.store_scatter(stage_d, [slot * 256 + iota * F + f], vals)
        stage_p[pl.ds(slot * L, L)] = tgtv
        pltpu.async_copy(stage_d.at[pl.ds(slot * 256, 256)],
                         rows_h.at[pl.ds(rbase + g * 256, 256)], ssem)
        pltpu.async_copy(stage_p.at[pl.ds(slot * L, L)],
                         pos_h.at[pl.ds(pbase + g * L, L)], ssem)

    lim = jnp.minimum(ngc, 2)

    def abody(g, c):
        ext(g)
        return c

    def bbody(g, c):
        wait_trio((g - k0) & 1)
        ext(g)
        return c

    lax.fori_loop(k0, k0 + lim, abody, 0)
    lax.fori_loop(k0 + lim, k0 + ngc, bbody, 0)
    return (lim, k0 + ngc)


def _phase(tab_h, ids_h, rows_h, pos_h, nwf, nch_max, tail0, tail_v,
           ids_v, mid_v, mpos_v, cmc_v, cmt_v, wbuf_v, stage_d, stage_p,
           dsem, ssem, wid, state):
    pltpu.sync_copy(ids_h, ids_v)
    wlo, whi = _wrange(wid, nwf)
    rbase = wid * (B * F)
    pbase = wid * B

    def fire(ci, slot):
        eff = pl.multiple_of(
            jnp.minimum(wlo + CW * ci, nwf - CW) * 128, 128)
        pltpu.async_copy(tab_h.at[:, pl.ds(eff, CU)], wbuf_v.at[slot], dsem)

    fire(0, 0)
    ngroups = _scan(ids_v, mid_v, mpos_v, wlo, whi)

    def chunk_body(c, st):
        cur = c & 1
        pltpu.make_async_copy(tab_h.at[:, pl.ds(0, CU)], wbuf_v.at[cur],
                              dsem).wait()
        fire(jnp.minimum(c + 1, nch_max - 1), 1 - cur)
        c0 = wlo + CW * c
        c1 = jnp.minimum(c0 + CW, nwf)
        eff = jnp.minimum(c0, nwf - CW) * 128
        ngc = _compact(mid_v, mpos_v, cmc_v, cmt_v, ngroups, c0, c1, eff)
        return _extract(cmc_v, cmt_v, ngc, wbuf_v.at[cur], rows_h, pos_h,
                        rbase, pbase, stage_d, stage_p, ssem, st, False)

    state = lax.fori_loop(0, nch_max, chunk_body, state)
    pltpu.make_async_copy(tab_h.at[:, pl.ds(0, CU)],
                          wbuf_v.at[nch_max & 1], dsem).wait()

    ngc = _compact(mid_v, mpos_v, cmc_v, cmt_v, ngroups, nwf, nwf + 1, tail0)
    return _extract(cmc_v, cmt_v, ngc, tail_v, rows_h, pos_h, rbase, pbase,
                    stage_d, stage_p, ssem, state, True)


@functools.partial(
    pl.kernel,
    out_type=[
        jax.ShapeDtypeStruct((NTILES * B * F,), jnp.float32),
        jax.ShapeDtypeStruct((NTILES * B,), jnp.int32),
        jax.ShapeDtypeStruct((NTILES * B * F,), jnp.float32),
        jax.ShapeDtypeStruct((NTILES * B,), jnp.int32),
        jax.ShapeDtypeStruct((2 * NTILES * L,), jnp.int32),
    ],
    mesh=_MESH,
    compiler_params=pltpu.CompilerParams(needs_layout_passes=False),
    scratch_types=[
        pltpu.VMEM((B,), jnp.int32),
        pltpu.VMEM((B + 4 * L,), jnp.int32),
        pltpu.VMEM((B + 4 * L,), jnp.int32),
        pltpu.VMEM((B + 4 * L,), jnp.int32),
        pltpu.VMEM((B + 4 * L,), jnp.int32),
        pltpu.VMEM((2, F, CU), jnp.float32),
        pltpu.VMEM((TAIL_U, 128), jnp.float32),
        pltpu.VMEM((512,), jnp.float32),
        pltpu.VMEM((2 * L,), jnp.int32),
        pltpu.VMEM((2 * L,), jnp.int32),
        pltpu.SemaphoreType.DMA,
        pltpu.SemaphoreType.DMA,
    ],
)
def _sc_gather(user_h, movie_h, ut_h, mt_h, tailu_h, tailm_h,
               rows_u_h, tgt_u_h, rows_m_h, tgt_m_h, cnt_h,
               ids_v, mid_v, mpos_v, cmc_v, cmt_v, wbuf_v, tail_v,
               stage_d, stage_p, cnt_v, dsem, ssem):
    wid = lax.axis_index("s") * 2 + lax.axis_index("c")
    pltpu.sync_copy(tailu_h, tail_v)
    st = _phase(ut_h, user_h, rows_u_h, tgt_u_h, NWF_U, NCH_U, TAIL_U0,
                tail_v, ids_v, mid_v, mpos_v, cmc_v, cmt_v, wbuf_v,
                stage_d, stage_p, dsem, ssem, wid, (0, 0))
    n_u = st[1]
    pltpu.sync_copy(tailm_h, tail_v.at[pl.ds(0, TAIL_M)])
    st = _phase(mt_h, movie_h, rows_m_h, tgt_m_h, NWF_M, NCH_M, TAIL_M0,
                tail_v, ids_v, mid_v, mpos_v, cmc_v, cmt_v, wbuf_v,
                stage_d, stage_p, dsem, ssem, wid, (st[0], 0))
    n_m = st[1]

    def drain(j, c):
        pltpu.make_async_copy(stage_d.at[pl.ds((j & 1) * 256, 256)],
                              rows_u_h.at[pl.ds(0, 256)], ssem).wait()
        pltpu.make_async_copy(stage_p.at[pl.ds((j & 1) * L, L)],
                              tgt_u_h.at[pl.ds(0, L)], ssem).wait()
        return c

    lax.fori_loop(0, st[0], drain, 0)

    cnt_v[pl.ds(0, L)] = jnp.full((L,), 1, jnp.int32) * n_u
    cnt_v[pl.ds(L, L)] = jnp.full((L,), 1, jnp.int32) * n_m
    pltpu.sync_copy(cnt_v.at[pl.ds(0, L)], cnt_h.at[pl.ds(wid * L, L)])
    pltpu.sync_copy(cnt_v.at[pl.ds(L, L)],
                    cnt_h.at[pl.ds(NTILES * L + wid * L, L)])


CG = 512


def _scatter_one(rows_h, tgt_h, out_h, n_groups, rbase, pbase,
                 rbuf, pbuf, dsem, ssem):
    def cbody(c, _):
        pltpu.sync_copy(rows_h.at[pl.ds(rbase + c * CG, CG)], rbuf)
        pltpu.sync_copy(tgt_h.at[pl.ds(pbase + c * CG, CG)],
                        pbuf.at[pl.ds(0, CG)])

        def sbody(i, _):
            sl = pl.ds(i * L, L)
            pbuf[sl] = pbuf[sl] >> 4
            return 0

        lax.fori_loop(0, CG // L, sbody, 0)
        rem = jnp.minimum(n_groups - c * (CG // L), CG // L)
        for i in range(7):
            pbuf[pl.ds(rem * L + i * L, L)] = jnp.full((L,), B, jnp.int32)
        nsc = (rem * L + 127) >> 7

        def fire(j, _):
            pltpu.async_copy(rbuf.at[pl.ds(j * 128, 128)],
                             out_h.at[pbuf.at[pl.ds(j * 128, 128)]], ssem)
            return 0

        def wait(j, _):
            pltpu.make_async_copy(rbuf.at[pl.ds(0, 128)],
                                  out_h.at[pbuf.at[pl.ds(0, 128)]],
                                  ssem).wait()
            return 0

        lax.fori_loop(0, nsc, fire, 0)
        lax.fori_loop(0, nsc, wait, 0)
        return 0

    nch = (n_groups + CG // L - 1) // (CG // L)
    lax.fori_loop(0, nch, cbody, 0)


@functools.partial(
    pl.kernel,
    out_type=[
        jax.ShapeDtypeStruct((B + L, F), jnp.float32),
        jax.ShapeDtypeStruct((B + L, F), jnp.float32),
    ],
    mesh=_MESH,
    compiler_params=pltpu.CompilerParams(needs_layout_passes=False,
                                         use_tc_tiling_on_sc=False),
    scratch_types=[
        pltpu.VMEM((CG, F), jnp.float32),
        pltpu.VMEM((CG + 128,), jnp.int32),
        pltpu.VMEM((2 * L,), jnp.int32),
        pltpu.SemaphoreType.DMA,
        pltpu.SemaphoreType.DMA,
    ],
)
def _sc_scatter(rows_u_h, tgt_u_h, rows_m_h, tgt_m_h, cnt_h, uo_h, mo_h,
                rbuf, pbuf, cnt_v, dsem, ssem):
    wid = lax.axis_index("s") * 2 + lax.axis_index("c")
    pltpu.sync_copy(cnt_h.at[pl.ds(wid * L, L)], cnt_v.at[pl.ds(0, L)])
    pltpu.sync_copy(cnt_h.at[pl.ds(NTILES * L + wid * L, L)],
                    cnt_v.at[pl.ds(L, L)])
    n_u = jnp.max(cnt_v[pl.ds(0, L)])
    n_m = jnp.max(cnt_v[pl.ds(L, L)])
    _scatter_one(rows_u_h, tgt_u_h, uo_h, n_u, wid * B, wid * B,
                 rbuf, pbuf, dsem, ssem)
    _scatter_one(rows_m_h, tgt_m_h, mo_h, n_m, wid * B, wid * B,
                 rbuf, pbuf, dsem, ssem)


def _mlp_body(u_ref, m_ref, w1u_ref, w1m_ref, b1_ref, w2_ref, b2_ref, o_ref):
    h = jnp.dot(u_ref[...], w1u_ref[...], preferred_element_type=jnp.float32)
    h = h + jnp.dot(m_ref[...], w1m_ref[...], preferred_element_type=jnp.float32)
    h = jnp.maximum(h + b1_ref[...], 0.0)
    o = jnp.dot(h, w2_ref[...], preferred_element_type=jnp.float32) + b2_ref[...]
    o_ref[...] = 5.5 / (1.0 + jnp.exp(-o))


def _mlp(u_pad, m_pad, w1u, w1m, b1, w2, b2):
    emb_spec = pl.BlockSpec((B, F), lambda i: (0, 0))

    def full(shape):
        return pl.BlockSpec(shape, lambda i: (0, 0))

    return pl.pallas_call(
        _mlp_body,
        grid=(1,),
        in_specs=[emb_spec, emb_spec, full((F, 64)), full((F, 64)),
                  full((1, 64)), full((64, 1)), full((1, 1))],
        out_specs=full((B, 1)),
        out_shape=jax.ShapeDtypeStruct((B, 1), jnp.float32),
    )(u_pad, m_pad, w1u, w1m, b1[None], w2, b2[None])


def kernel(user, movie, u_table, m_table, W1, b1, W2, b2):
    user = user.astype(jnp.int32)
    movie = movie.astype(jnp.int32)
    pad = ((0, 0), (0, 128 - F))
    tailu = jnp.pad(u_table[TAIL_U0:], pad)
    tailm = jnp.pad(m_table[TAIL_M0:], pad)
    rows_u, tgt_u, rows_m, tgt_m, cnt = _sc_gather(
        user, movie, u_table.T, m_table.T, tailu, tailm)
    uo, mo = _sc_scatter(rows_u.reshape(NTILES * B, F), tgt_u,
                         rows_m.reshape(NTILES * B, F), tgt_m, cnt)
    return _mlp(uo, mo, W1[:F], W1[F:], b1, W2, b2)


# speedup vs baseline: 118.3829x; 1.0497x over previous
import functools

import jax
import jax.numpy as jnp
from jax import lax
from jax.experimental import pallas as pl
from jax.experimental.pallas import tpu as pltpu
from jax.experimental.pallas import tpu_sc as plsc

B = 16384
F = 16
NU = 1000000
NM = 100000
L = 16
NTILES = 32
CW = 8
CU = CW * 128

NWF_U = NU // 128
NWF_M = NM // 128
TAIL_U0 = NWF_U * 128
TAIL_M0 = NWF_M * 128
TAIL_U = NU - TAIL_U0
TAIL_M = NM - TAIL_M0
NCH_U = 31
NCH_M = 4
SAFE = B * F

_MESH = plsc.VectorSubcoreMesh(core_axis_name="c", subcore_axis_name="s")


def _wrange(wid, nwf):
    per = nwf // NTILES
    rem = nwf - per * NTILES
    wlo = wid * per + jnp.minimum(wid, rem)
    cnt = per + (wid < rem).astype(jnp.int32)
    whi = wlo + cnt + (wid == NTILES - 1).astype(jnp.int32)
    return wlo, whi


def _scan(ids_v, mid_v, mpos_v, wlo, whi):
    def body(q, n):
        iota = lax.iota(jnp.int32, L)
        parts = []
        for j in range(4):
            g = q * 4 + j
            idv = ids_v[pl.ds(g * L, L)]
            w = idv >> 7
            msk = (w >= wlo) & (w < whi)
            parts.append((idv, g * L + iota, msk,
                          jnp.sum(msk.astype(jnp.int32))))
        for idv, posv, msk, s in parts:
            plsc.store_compressed(mid_v.at[pl.ds(n, L)], idv, mask=msk)
            plsc.store_compressed(mpos_v.at[pl.ds(n, L)], posv, mask=msk)
            n = n + s
        return n

    n = lax.fori_loop(0, B // L // 4, body, 0)
    for j in range(4):
        mid_v[pl.ds(n + j * L, L)] = jnp.full((L,), -1, jnp.int32)
    return (n + L - 1) >> 4


def _compact(mid_v, mpos_v, cmc_v, cmt_v, ngroups, lo_w, hi_w, col_base):
    def body(q, nc):
        parts = []
        for j in range(4):
            g = q * 4 + j
            idv = mid_v[pl.ds(g * L, L)]
            posv = mpos_v[pl.ds(g * L, L)]
            w = idv >> 7
            msk = (w >= lo_w) & (w < hi_w)
            parts.append((idv, posv, msk, jnp.sum(msk.astype(jnp.int32))))
        for idv, posv, msk, s in parts:
            plsc.store_compressed(cmc_v.at[pl.ds(nc, L)], idv - col_base,
                                  mask=msk)
            plsc.store_compressed(cmt_v.at[pl.ds(nc, L)], posv, mask=msk)
            nc = nc + s
        return nc

    nc = lax.fori_loop(0, (ngroups + 3) >> 2, body, 0)
    cmc_v[pl.ds(nc, L)] = jnp.full((L,), 0, jnp.int32)
    cmt_v[pl.ds(nc, L)] = B + lax.iota(jnp.int32, L)
    return (nc + L - 1) >> 4


def _extract(cmc_v, cmt_v, ngc, src_v, out_h, stage_d, stage_p,
             ssem, state, row_is_id):
    prev_out, k0 = state

    def wait_one(slot):
        pltpu.make_async_copy(stage_d.at[slot],
                              out_h.at[stage_p.at[slot]], ssem).wait()

    def drain(j, c):
        wait_one(j & 1)
        return c

    lax.fori_loop(0, prev_out, drain, 0)

    def ext(g):
        slot = (g - k0) & 1
        slotv = jnp.full((L,), 1, jnp.int32) * slot
        colv = cmc_v[pl.ds((g - k0) * L, L)]
        tgtv = cmt_v[pl.ds((g - k0) * L, L)]
        iota = lax.iota(jnp.int32, L)
        for f in range(F):
            fv = jnp.full((L,), f, jnp.int32)
            if row_is_id:
                vals = plsc.load_gather(src_v, [colv, fv])
            else:
                vals = plsc.load_gather(src_v, [fv, colv])
            plsc.store_scatter(stage_d, [slotv, iota, fv], vals)
        plsc.store_scatter(stage_p, [slotv, iota], tgtv)
        pltpu.async_copy(stage_d.at[slot], out_h.at[stage_p.at[slot]], ssem)

    lim = jnp.minimum(ngc, 2)

    def abody(g, c):
        ext(g)
        return c

    def bbody(g, c):
        wait_one((g - k0) & 1)
        ext(g)
        return c

    lax.fori_loop(k0, k0 + lim, abody, 0)
    lax.fori_loop(k0 + lim, k0 + ngc, bbody, 0)
    return (lim, k0 + ngc)


def _phase(tab_h, ids_h, out_h, nwf, nch_max, tail0, tail_v,
           ids_v, mid_v, mpos_v, cmc_v, cmt_v, wbuf_v, stage_d, stage_p,
           dsem, ssem, wid, state):
    pltpu.sync_copy(ids_h, ids_v)
    wlo, whi = _wrange(wid, nwf)

    def fire(ci, slot):
        eff = pl.multiple_of(
            jnp.minimum(wlo + CW * ci, nwf - CW) * 128, 128)
        pltpu.async_copy(tab_h.at[:, pl.ds(eff, CU)], wbuf_v.at[slot], dsem)

    fire(0, 0)
    ngroups = _scan(ids_v, mid_v, mpos_v, wlo, whi)

    def chunk_body(c, st):
        cur = c & 1
        pltpu.make_async_copy(tab_h.at[:, pl.ds(0, CU)], wbuf_v.at[cur],
                              dsem).wait()
        fire(jnp.minimum(c + 1, nch_max - 1), 1 - cur)
        c0 = wlo + CW * c
        c1 = jnp.minimum(c0 + CW, nwf)
        eff = jnp.minimum(c0, nwf - CW) * 128
        ngc = _compact(mid_v, mpos_v, cmc_v, cmt_v, ngroups, c0, c1, eff)
        return _extract(cmc_v, cmt_v, ngc, wbuf_v.at[cur], out_h,
                        stage_d, stage_p, ssem, st, False)

    state = lax.fori_loop(0, nch_max, chunk_body, state)
    pltpu.make_async_copy(tab_h.at[:, pl.ds(0, CU)],
                          wbuf_v.at[nch_max & 1], dsem).wait()

    ngc = _compact(mid_v, mpos_v, cmc_v, cmt_v, ngroups, nwf, nwf + 1, tail0)
    return _extract(cmc_v, cmt_v, ngc, tail_v, out_h,
                    stage_d, stage_p, ssem, state, True)


@functools.partial(
    pl.kernel,
    out_type=[
        jax.ShapeDtypeStruct((B + L, 128), jnp.float32),
        jax.ShapeDtypeStruct((B + L, 128), jnp.float32),
    ],
    mesh=_MESH,
    compiler_params=pltpu.CompilerParams(needs_layout_passes=False),
    scratch_types=[
        pltpu.VMEM((B,), jnp.int32),
        pltpu.VMEM((B + 4 * L,), jnp.int32),
        pltpu.VMEM((B + 4 * L,), jnp.int32),
        pltpu.VMEM((B + 4 * L,), jnp.int32),
        pltpu.VMEM((B + 4 * L,), jnp.int32),
        pltpu.VMEM((2, F, CU), jnp.float32),
        pltpu.VMEM((TAIL_U, 128), jnp.float32),
        pltpu.VMEM((2, L, 128), jnp.float32),
        pltpu.VMEM((2, L), jnp.int32),
        pltpu.SemaphoreType.DMA,
        pltpu.SemaphoreType.DMA,
    ],
)
def _sc_gather(user_h, movie_h, ut_h, mt_h, tailu_h, tailm_h, uo_h, mo_h,
               ids_v, mid_v, mpos_v, cmc_v, cmt_v, wbuf_v, tail_v,
               stage_d, stage_p, dsem, ssem):
    wid = lax.axis_index("s") * 2 + lax.axis_index("c")
    pltpu.sync_copy(tailu_h, tail_v)
    st = _phase(ut_h, user_h, uo_h, NWF_U, NCH_U, TAIL_U0,
                tail_v, ids_v, mid_v, mpos_v, cmc_v, cmt_v, wbuf_v,
                stage_d, stage_p, dsem, ssem, wid, (0, 0))
    pltpu.sync_copy(tailm_h, tail_v.at[pl.ds(0, TAIL_M)])
    st = _phase(mt_h, movie_h, mo_h, NWF_M, NCH_M, TAIL_M0,
                tail_v, ids_v, mid_v, mpos_v, cmc_v, cmt_v, wbuf_v,
                stage_d, stage_p, dsem, ssem, wid, (st[0], 0))

    def drain(j, c):
        pltpu.make_async_copy(stage_d.at[j & 1],
                              mo_h.at[stage_p.at[j & 1]], ssem).wait()
        return c

    lax.fori_loop(0, st[0], drain, 0)


def _mlp_body(u_ref, m_ref, w1u_ref, w1m_ref, b1_ref, w2_ref, b2_ref, o_ref):
    u = u_ref[...][:, :F]
    m = m_ref[...][:, :F]
    h = jnp.dot(u, w1u_ref[...], preferred_element_type=jnp.float32)
    h = h + jnp.dot(m, w1m_ref[...], preferred_element_type=jnp.float32)
    h = jnp.maximum(h + b1_ref[...], 0.0)
    o = jnp.dot(h, w2_ref[...], preferred_element_type=jnp.float32) + b2_ref[...]
    o_ref[...] = 5.5 / (1.0 + jnp.exp(-o))


def _mlp(u_pad, m_pad, w1u, w1m, b1, w2, b2):
    emb_spec = pl.BlockSpec((B, 128), lambda i: (0, 0))

    def full(shape):
        return pl.BlockSpec(shape, lambda i: (0, 0))

    return pl.pallas_call(
        _mlp_body,
        grid=(1,),
        in_specs=[emb_spec, emb_spec, full((F, 64)), full((F, 64)),
                  full((1, 64)), full((64, 1)), full((1, 1))],
        out_specs=full((B, 1)),
        out_shape=jax.ShapeDtypeStruct((B, 1), jnp.float32),
    )(u_pad, m_pad, w1u, w1m, b1[None], w2, b2[None])


def kernel(user, movie, u_table, m_table, W1, b1, W2, b2):
    user = user.astype(jnp.int32)
    movie = movie.astype(jnp.int32)
    pad = ((0, 0), (0, 128 - F))
    tailu = jnp.pad(u_table[TAIL_U0:], pad)
    tailm = jnp.pad(m_table[TAIL_M0:], pad)
    uo, mo = _sc_gather(user, movie, u_table.T, m_table.T, tailu, tailm)
    return _mlp(uo, mo, W1[:F], W1[F:], b1, W2, b2)


# speedup vs baseline: 118.7480x; 1.0031x over previous
import functools

import jax
import jax.numpy as jnp
from jax import lax
from jax.experimental import pallas as pl
from jax.experimental.pallas import tpu as pltpu
from jax.experimental.pallas import tpu_sc as plsc

B = 16384
F = 16
NU = 1000000
NM = 100000
L = 16
NTILES = 32
CW = 8
CU = CW * 128

NWF_U = NU // 128
NWF_M = NM // 128
TAIL_U0 = NWF_U * 128
TAIL_M0 = NWF_M * 128
TAIL_U = NU - TAIL_U0
TAIL_M = NM - TAIL_M0
NCH_U = 31
NCH_M = 4
SAFE = B * F

_MESH = plsc.VectorSubcoreMesh(core_axis_name="c", subcore_axis_name="s")


def _wrange(wid, nwf):
    per = nwf // NTILES
    rem = nwf - per * NTILES
    wlo = wid * per + jnp.minimum(wid, rem)
    cnt = per + (wid < rem).astype(jnp.int32)
    whi = wlo + cnt + (wid == NTILES - 1).astype(jnp.int32)
    return wlo, whi


def _scan(ids_v, mid_v, mpos_v, wlo, whi):
    def body(q, n):
        iota = lax.iota(jnp.int32, L)
        parts = []
        for j in range(4):
            g = q * 4 + j
            idv = ids_v[pl.ds(g * L, L)]
            w = idv >> 7
            msk = (w >= wlo) & (w < whi)
            parts.append((idv, g * L + iota, msk,
                          jnp.sum(msk.astype(jnp.int32))))
        for idv, posv, msk, s in parts:
            plsc.store_compressed(mid_v.at[pl.ds(n, L)], idv, mask=msk)
            plsc.store_compressed(mpos_v.at[pl.ds(n, L)], posv, mask=msk)
            n = n + s
        return n

    n = lax.fori_loop(0, B // L // 4, body, 0)
    for j in range(4):
        mid_v[pl.ds(n + j * L, L)] = jnp.full((L,), -1, jnp.int32)
    return (n + L - 1) >> 4


def _compact(mid_v, mpos_v, cmc_v, cmt_v, ngroups, lo_w, hi_w, col_base):
    def body(q, nc):
        parts = []
        for j in range(4):
            g = q * 4 + j
            idv = mid_v[pl.ds(g * L, L)]
            posv = mpos_v[pl.ds(g * L, L)]
            w = idv >> 7
            msk = (w >= lo_w) & (w < hi_w)
            parts.append((idv, posv, msk, jnp.sum(msk.astype(jnp.int32))))
        for idv, posv, msk, s in parts:
            plsc.store_compressed(cmc_v.at[pl.ds(nc, L)], idv - col_base,
                                  mask=msk)
            plsc.store_compressed(cmt_v.at[pl.ds(nc, L)], posv, mask=msk)
            nc = nc + s
        return nc

    nc = lax.fori_loop(0, (ngroups + 3) >> 2, body, 0)
    cmc_v[pl.ds(nc, L)] = jnp.full((L,), 0, jnp.int32)
    cmt_v[pl.ds(nc, L)] = B + lax.iota(jnp.int32, L)
    return (nc + L - 1) >> 4


def _extract(cmc_v, cmt_v, ngc, src_v, out_h, stage_d, stage_p,
             ssem, state, row_is_id):
    prev_out, k0 = state

    def wait_one(slot):
        pltpu.make_async_copy(stage_d.at[slot],
                              out_h.at[stage_p.at[slot]], ssem).wait()

    def drain(j, c):
        wait_one(j & 3)
        return c

    lax.fori_loop(0, prev_out, drain, 0)

    def ext(g):
        slot = (g - k0) & 3
        slotv = jnp.full((L,), 1, jnp.int32) * slot
        colv = cmc_v[pl.ds((g - k0) * L, L)]
        tgtv = cmt_v[pl.ds((g - k0) * L, L)]
        iota = lax.iota(jnp.int32, L)
        for f in range(F):
            fv = jnp.full((L,), f, jnp.int32)
            if row_is_id:
                tv = colv * F + f
                vals = plsc.load_gather(src_v, [tv >> 7, tv & 127])
            else:
                vals = plsc.load_gather(src_v, [fv, colv])
            plsc.store_scatter(stage_d, [slotv, iota, fv], vals)
        plsc.store_scatter(stage_p, [slotv, iota], tgtv)
        pltpu.async_copy(stage_d.at[slot], out_h.at[stage_p.at[slot]], ssem)

    lim = jnp.minimum(ngc, 4)

    def abody(g, c):
        ext(g)
        return c

    def bbody(g, c):
        wait_one((g - k0) & 3)
        ext(g)
        return c

    lax.fori_loop(k0, k0 + lim, abody, 0)
    lax.fori_loop(k0 + lim, k0 + ngc, bbody, 0)
    return (lim, k0 + ngc)


def _phase(tab_h, ids_h, out_h, nwf, nch_max, tail0, tail_v,
           ids_v, mid_v, mpos_v, cmc_v, cmt_v, wbuf_v, stage_d, stage_p,
           dsem, ssem, wid, state):
    pltpu.sync_copy(ids_h, ids_v)
    wlo, whi = _wrange(wid, nwf)

    def fire(ci, slot):
        eff = pl.multiple_of(
            jnp.minimum(wlo + CW * ci, nwf - CW) * 128, 128)
        pltpu.async_copy(tab_h.at[:, pl.ds(eff, CU)], wbuf_v.at[slot], dsem)

    fire(0, 0)
    ngroups = _scan(ids_v, mid_v, mpos_v, wlo, whi)

    def chunk_body(c, st):
        cur = c & 1
        pltpu.make_async_copy(tab_h.at[:, pl.ds(0, CU)], wbuf_v.at[cur],
                              dsem).wait()
        fire(jnp.minimum(c + 1, nch_max - 1), 1 - cur)
        c0 = wlo + CW * c
        c1 = jnp.minimum(c0 + CW, nwf)
        eff = jnp.minimum(c0, nwf - CW) * 128
        ngc = _compact(mid_v, mpos_v, cmc_v, cmt_v, ngroups, c0, c1, eff)
        return _extract(cmc_v, cmt_v, ngc, wbuf_v.at[cur], out_h,
                        stage_d, stage_p, ssem, st, False)

    state = lax.fori_loop(0, nch_max, chunk_body, state)
    pltpu.make_async_copy(tab_h.at[:, pl.ds(0, CU)],
                          wbuf_v.at[nch_max & 1], dsem).wait()

    ngc = _compact(mid_v, mpos_v, cmc_v, cmt_v, ngroups, nwf, nwf + 1, tail0)
    return _extract(cmc_v, cmt_v, ngc, tail_v, out_h,
                    stage_d, stage_p, ssem, state, True)


@functools.partial(
    pl.kernel,
    out_type=[
        jax.ShapeDtypeStruct((B + L, 128), jnp.float32),
        jax.ShapeDtypeStruct((B + L, 128), jnp.float32),
    ],
    mesh=_MESH,
    compiler_params=pltpu.CompilerParams(needs_layout_passes=False),
    scratch_types=[
        pltpu.VMEM((B,), jnp.int32),
        pltpu.VMEM((B + 4 * L,), jnp.int32),
        pltpu.VMEM((B + 4 * L,), jnp.int32),
        pltpu.VMEM((B + 4 * L,), jnp.int32),
        pltpu.VMEM((B + 4 * L,), jnp.int32),
        pltpu.VMEM((2, F, CU), jnp.float32),
        pltpu.VMEM((8, 128), jnp.float32),
        pltpu.VMEM((4, L, 128), jnp.float32),
        pltpu.VMEM((4, L), jnp.int32),
        pltpu.SemaphoreType.DMA,
        pltpu.SemaphoreType.DMA,
    ],
)
def _sc_gather(user_h, movie_h, ut_h, mt_h, tailu_h, tailm_h, uo_h, mo_h,
               ids_v, mid_v, mpos_v, cmc_v, cmt_v, wbuf_v, tail_v,
               stage_d, stage_p, dsem, ssem):
    wid = lax.axis_index("s") * 2 + lax.axis_index("c")
    pltpu.sync_copy(tailu_h, tail_v)
    st = _phase(ut_h, user_h, uo_h, NWF_U, NCH_U, TAIL_U0,
                tail_v, ids_v, mid_v, mpos_v, cmc_v, cmt_v, wbuf_v,
                stage_d, stage_p, dsem, ssem, wid, (0, 0))
    pltpu.sync_copy(tailm_h, tail_v.at[pl.ds(0, 4)])
    st = _phase(mt_h, movie_h, mo_h, NWF_M, NCH_M, TAIL_M0,
                tail_v, ids_v, mid_v, mpos_v, cmc_v, cmt_v, wbuf_v,
                stage_d, stage_p, dsem, ssem, wid, (st[0], 0))

    def drain(j, c):
        pltpu.make_async_copy(stage_d.at[j & 3],
                              mo_h.at[stage_p.at[j & 3]], ssem).wait()
        return c

    lax.fori_loop(0, st[0], drain, 0)


def _mlp_body(u_ref, m_ref, w1u_ref, w1m_ref, b1_ref, w2_ref, b2_ref, o_ref):
    u = u_ref[...][:, :F]
    m = m_ref[...][:, :F]
    h = jnp.dot(u, w1u_ref[...], preferred_element_type=jnp.float32)
    h = h + jnp.dot(m, w1m_ref[...], preferred_element_type=jnp.float32)
    h = jnp.maximum(h + b1_ref[...], 0.0)
    o = jnp.dot(h, w2_ref[...], preferred_element_type=jnp.float32) + b2_ref[...]
    o_ref[...] = 5.5 / (1.0 + jnp.exp(-o))


def _mlp(u_pad, m_pad, w1u, w1m, b1, w2, b2):
    emb_spec = pl.BlockSpec((B, 128), lambda i: (0, 0))

    def full(shape):
        return pl.BlockSpec(shape, lambda i: (0, 0))

    return pl.pallas_call(
        _mlp_body,
        grid=(1,),
        in_specs=[emb_spec, emb_spec, full((F, 64)), full((F, 64)),
                  full((1, 64)), full((64, 1)), full((1, 1))],
        out_specs=full((B, 1)),
        out_shape=jax.ShapeDtypeStruct((B, 1), jnp.float32),
    )(u_pad, m_pad, w1u, w1m, b1[None], w2, b2[None])


def kernel(user, movie, u_table, m_table, W1, b1, W2, b2):
    user = user.astype(jnp.int32)
    movie = movie.astype(jnp.int32)
    tailu = u_table[TAIL_U0:].reshape(8, 128)
    tailm = m_table[TAIL_M0:].reshape(4, 128)
    uo, mo = _sc_gather(user, movie, u_table.T, m_table.T, tailu, tailm)
    return _mlp(uo, mo, W1[:F], W1[F:], b1, W2, b2)
